# baseline (speedup 1.0000x reference)
import functools

import jax
import jax.numpy as jnp
from jax import lax
from jax.experimental import pallas as pl
from jax.experimental.pallas import tpu as pltpu
from jax.experimental.pallas import tpu_sc as plsc

N_CAT = 26
CAT_CARD = 100000
EMB_DIM = 16
HIDDEN = 128
F = N_CAT * EMB_DIM

NUM_CORES = 2
NUM_SUBCORES = 16
NUM_WORKERS = NUM_CORES * NUM_SUBCORES
ROWS_PER_W = F // NUM_WORKERS


def _sc_gather_t(t3, idx3, n_rb, f0, nf):
    mesh = plsc.VectorSubcoreMesh(core_axis_name="c", subcore_axis_name="s")
    NQ = 4
    Q = n_rb // NQ
    rpw = nf // NUM_WORKERS

    @functools.partial(
        pl.kernel,
        out_type=jax.ShapeDtypeStruct((nf, n_rb, 128), jnp.float32),
        mesh=mesh,
        compiler_params=pltpu.CompilerParams(
            use_tc_tiling_on_sc=True, needs_layout_passes=False
        ),
        scratch_types=[
            pltpu.VMEM((CAT_CARD,), jnp.float32),
            pltpu.VMEM((2, Q, 128), jnp.int32),
            pltpu.VMEM((2, Q, 128), jnp.float32),
            pltpu.SemaphoreType.DMA,
            pltpu.SemaphoreType.DMA,
            pltpu.SemaphoreType.DMA,
        ],
    )
    def k(t_hbm, idx_hbm, out_hbm, row_v, idx_v, out_v, rsem, isem, osem):
        w = lax.axis_index("s") * NUM_CORES + lax.axis_index("c")

        def row_copy(j):
            f = f0 + w * rpw + j
            return pltpu.async_copy(
                t_hbm.at[f // EMB_DIM, f % EMB_DIM], row_v, rsem
            )

        def idx_copy(j, q, s):
            col = (f0 + w * rpw + j) // EMB_DIM
            return pltpu.async_copy(
                idx_hbm.at[col, pl.ds(q * Q, Q)], idx_v.at[s], isem
            )

        rcp = row_copy(0)
        icp = idx_copy(0, 0, 0)
        ocp = [None, None]
        for j in range(rpw):
            fl = w * rpw + j
            rcp.wait()
            for q in range(NQ):
                s = q & 1
                icp.wait()
                if q < NQ - 1:
                    icp = idx_copy(j, q + 1, 1 - s)
                elif j < rpw - 1:
                    icp = idx_copy(j + 1, 0, 1 - s)
                if ocp[s] is not None:
                    ocp[s].wait()

                @plsc.parallel_loop(0, Q, 1, unroll=1)
                def _(rr, s=s):
                    for u in range(8):
                        vidx = idx_v[s, rr, pl.ds(u * 16, 16)]
                        out_v[s, rr, pl.ds(u * 16, 16)] = plsc.load_gather(
                            row_v, [vidx]
                        )
                if q == NQ - 1 and j < rpw - 1:
                    rcp = row_copy(j + 1)
                ocp[s] = pltpu.async_copy(
                    out_v.at[s], out_hbm.at[fl, pl.ds(q * Q, Q)], osem
                )
        for cp in ocp:
            if cp is not None:
                cp.wait()

    return k(t3, idx3)


def _tc_matmul_t(embsT3, cont, W1, W2, b2, bs, ms, nr):
    FB = embsT3.shape[0]
    n_rb = embsT3.shape[1]
    BR = nr * bs
    RB = BR // 128
    C = cont.shape[0]

    def mm(e_ref, c_ref, w1_ref, w2_ref, b_ref, o_ref):
        e = e_ref[...].reshape(FB, BR)
        acc = lax.dot_general(
            e, w1_ref[...], (((0,), (0,)), ((), ())),
            preferred_element_type=jnp.float32,
        )
        acc = acc + lax.dot_general(
            c_ref[...], w2_ref[...], (((0,), (0,)), ((), ())),
            preferred_element_type=jnp.float32,
        )
        acc = acc + b_ref[...]
        o_ref[...] = acc.reshape(nr, bs, HIDDEN).transpose(1, 0, 2).reshape(
            bs, 1, nr, HIDDEN
        )

    return pl.pallas_call(
        mm,
        grid=(ms,),
        in_specs=[
            pl.BlockSpec((FB, RB, 128), lambda i: (0, i, 0)),
            pl.BlockSpec((C, BR), lambda i: (0, i)),
            pl.BlockSpec((FB, HIDDEN), lambda i: (0, 0)),
            pl.BlockSpec((C, HIDDEN), lambda i: (0, 0)),
            pl.BlockSpec((1, HIDDEN), lambda i: (0, 0)),
        ],
        out_specs=pl.BlockSpec((bs, 1, nr, HIDDEN), lambda i: (0, i, 0, 0)),
        out_shape=jax.ShapeDtypeStruct((bs, ms, nr, HIDDEN), jnp.float32),
    )(embsT3, cont, W1, W2, b2)


def _tc_matmul_acc(embsT3, partial, W1b, bs, ms, nr):
    FB = embsT3.shape[0]
    BR = nr * bs
    RB = BR // 128

    def mm(e_ref, p_ref, w_ref, o_ref):
        e = e_ref[...].reshape(FB, BR)
        acc = lax.dot_general(
            e, w_ref[...], (((0,), (0,)), ((), ())),
            preferred_element_type=jnp.float32,
        )
        o_ref[...] = p_ref[...] + acc.reshape(nr, bs, HIDDEN).transpose(
            1, 0, 2
        ).reshape(bs, 1, nr, HIDDEN)

    return pl.pallas_call(
        mm,
        grid=(ms,),
        in_specs=[
            pl.BlockSpec((FB, RB, 128), lambda i: (0, i, 0)),
            pl.BlockSpec((bs, 1, nr, HIDDEN), lambda i: (0, i, 0, 0)),
            pl.BlockSpec((FB, HIDDEN), lambda i: (0, 0)),
        ],
        out_specs=pl.BlockSpec((bs, 1, nr, HIDDEN), lambda i: (0, i, 0, 0)),
        out_shape=jax.ShapeDtypeStruct((bs, ms, nr, HIDDEN), jnp.float32),
        input_output_aliases={1: 0},
    )(embsT3, partial, W1b)


def kernel(x, tables, W, b):
    bs, ms, nr, d = x.shape
    R = bs * ms * nr
    n_rb = R // 128
    t3 = tables.transpose(0, 2, 1)
    xt = x.transpose(1, 2, 3, 0)
    idx3 = (
        xt[:, :, :N_CAT, :]
        .astype(jnp.int32)
        .transpose(2, 0, 1, 3)
        .reshape(N_CAT, n_rb, 128)
    )
    cont = xt[:, :, N_CAT:, :].transpose(2, 0, 1, 3).reshape(d - N_CAT, R)
    FA = 224
    embsA = _sc_gather_t(t3, idx3, n_rb, 0, FA)
    embsB = _sc_gather_t(t3, idx3, n_rb, FA, F - FA)
    b2 = b.reshape(1, HIDDEN)
    part = _tc_matmul_t(embsA, cont, W[:FA], W[F:], b2, bs, ms, nr)
    return _tc_matmul_acc(embsB, part, W[FA:F], bs, ms, nr)


# speedup vs baseline: 1.0510x; 1.0510x over previous
import functools

import jax
import jax.numpy as jnp
from jax import lax
from jax.experimental import pallas as pl
from jax.experimental.pallas import tpu as pltpu
from jax.experimental.pallas import tpu_sc as plsc

N_CAT = 26
CAT_CARD = 100000
EMB_DIM = 16
HIDDEN = 128
F = N_CAT * EMB_DIM

NUM_CORES = 2
NUM_SUBCORES = 16
NUM_WORKERS = NUM_CORES * NUM_SUBCORES
ROWS_PER_W = F // NUM_WORKERS


def _sc_gather_t(t3, idx3, n_rb):
    mesh = plsc.VectorSubcoreMesh(core_axis_name="c", subcore_axis_name="s")
    NQ = 4
    Q = n_rb // NQ

    @functools.partial(
        pl.kernel,
        out_type=jax.ShapeDtypeStruct((F, n_rb, 128), jnp.float32),
        mesh=mesh,
        compiler_params=pltpu.CompilerParams(
            use_tc_tiling_on_sc=True, needs_layout_passes=False
        ),
        scratch_types=[
            pltpu.VMEM((CAT_CARD,), jnp.float32),
            pltpu.VMEM((2, Q, 128), jnp.int32),
            pltpu.VMEM((2, Q, 128), jnp.float32),
            pltpu.SemaphoreType.DMA,
            pltpu.SemaphoreType.DMA,
            pltpu.SemaphoreType.DMA,
        ],
    )
    def k(t_hbm, idx_hbm, out_hbm, row_v, idx_v, out_v, rsem, isem, osem):
        w = lax.axis_index("s") * NUM_CORES + lax.axis_index("c")

        def row_copy(j):
            f = w * ROWS_PER_W + j
            return [
                pltpu.async_copy(
                    t_hbm.at[f // EMB_DIM, f % EMB_DIM], row_v, rsem
                )
            ]

        def idx_copy(j, q, s):
            col = (w * ROWS_PER_W + j) // EMB_DIM
            return pltpu.async_copy(
                idx_hbm.at[col, pl.ds(q * Q, Q)], idx_v.at[s], isem
            )

        rcp = row_copy(0)
        icp = idx_copy(0, 0, 0)
        ocp = [None, None]
        for j in range(ROWS_PER_W):
            f = w * ROWS_PER_W + j
            for cp in rcp:
                cp.wait()
            for q in range(NQ):
                s = q & 1
                icp.wait()
                if q < NQ - 1:
                    icp = idx_copy(j, q + 1, 1 - s)
                elif j < ROWS_PER_W - 1:
                    icp = idx_copy(j + 1, 0, 1 - s)
                if ocp[s] is not None:
                    ocp[s].wait()

                @plsc.parallel_loop(0, Q, 1, unroll=1)
                def _(rr, s=s):
                    for u in range(8):
                        vidx = idx_v[s, rr, pl.ds(u * 16, 16)]
                        out_v[s, rr, pl.ds(u * 16, 16)] = plsc.load_gather(
                            row_v, [vidx]
                        )
                if q == NQ - 1 and j < ROWS_PER_W - 1:
                    rcp = row_copy(j + 1)
                ocp[s] = pltpu.async_copy(
                    out_v.at[s], out_hbm.at[f, pl.ds(q * Q, Q)], osem
                )
        for cp in ocp:
            if cp is not None:
                cp.wait()

    return k(t3, idx3)


def _tc_matmul_t(embsT3, cont, W1, W2, b2, bs, ms, nr):
    n_rb = embsT3.shape[1]
    BR = nr * bs
    RB = BR // 128
    C = cont.shape[0]

    def mm(e_ref, c_ref, w1_ref, w2_ref, b_ref, o_ref):
        e = e_ref[...].reshape(F, BR)
        acc = lax.dot_general(
            e, w1_ref[...], (((0,), (0,)), ((), ())),
            preferred_element_type=jnp.float32,
        )
        acc = acc + lax.dot_general(
            c_ref[...], w2_ref[...], (((0,), (0,)), ((), ())),
            preferred_element_type=jnp.float32,
        )
        acc = acc + b_ref[...]
        o_ref[...] = acc.reshape(nr, bs, HIDDEN).transpose(1, 0, 2).reshape(
            bs, 1, nr, HIDDEN
        )

    return pl.pallas_call(
        mm,
        grid=(ms,),
        in_specs=[
            pl.BlockSpec((F, RB, 128), lambda i: (0, i, 0)),
            pl.BlockSpec((C, BR), lambda i: (0, i)),
            pl.BlockSpec((F, HIDDEN), lambda i: (0, 0)),
            pl.BlockSpec((C, HIDDEN), lambda i: (0, 0)),
            pl.BlockSpec((1, HIDDEN), lambda i: (0, 0)),
        ],
        out_specs=pl.BlockSpec((bs, 1, nr, HIDDEN), lambda i: (0, i, 0, 0)),
        out_shape=jax.ShapeDtypeStruct((bs, ms, nr, HIDDEN), jnp.float32),
    )(embsT3, cont, W1, W2, b2)


def kernel(x, tables, W, b):
    bs, ms, nr, d = x.shape
    R = bs * ms * nr
    n_rb = R // 128
    t3 = tables.transpose(0, 2, 1)
    xt = x.transpose(1, 2, 3, 0)
    idx3 = (
        xt[:, :, :N_CAT, :]
        .astype(jnp.int32)
        .transpose(2, 0, 1, 3)
        .reshape(N_CAT, n_rb, 128)
    )
    cont = xt[:, :, N_CAT:, :].transpose(2, 0, 1, 3).reshape(d - N_CAT, R)
    embsT3 = _sc_gather_t(t3, idx3, n_rb)
    W1 = W[:F]
    W2 = W[F:]
    return _tc_matmul_t(embsT3, cont, W1, W2, b.reshape(1, HIDDEN), bs, ms, nr)
